# Initial kernel scaffold; baseline (speedup 1.0000x reference)
import jax
import jax.numpy as jnp
from jax.experimental import pallas as pl


def kernel(tokens, table):
    raise NotImplementedError("write your pallas kernel here")


# speedup vs baseline: 1.1877x; 1.1877x over previous
import math

import jax
import jax.numpy as jnp
from jax import lax
from jax.experimental import pallas as pl
from jax.experimental.pallas import tpu as pltpu
from jax.experimental.pallas import tpu_sc as plsc

EMB = 32
SCALE = math.sqrt(EMB)
LANES = 16
CH = 128
NCH = 8
GROUP = CH * NCH


def _build(n_rows128, nc, ns):
    nw = nc * ns
    rows_per_worker = n_rows128 // nw
    n_groups = rows_per_worker // NCH

    mesh = plsc.VectorSubcoreMesh(core_axis_name="c", subcore_axis_name="s",
                                  num_cores=nc, num_subcores=ns)

    def body(idx_hbm, table_hbm, out_hbm, idx_v, rows_v, gsem):
        w = lax.axis_index("s") * nc + lax.axis_index("c")
        row0 = w * rows_per_worker

        def group(g, carry):
            base = row0 + g * NCH
            pltpu.sync_copy(idx_hbm.at[pl.ds(base, NCH)], idx_v)
            descs = [
                pltpu.async_copy(table_hbm.at[idx_v.at[j]], rows_v.at[j], gsem)
                for j in range(NCH)
            ]
            for d in descs:
                d.wait()

            def scale_row(i, c2):
                for j in range(NCH):
                    rows_v[j, i, 0:LANES] = rows_v[j, i, 0:LANES] * SCALE
                    rows_v[j, i, LANES:EMB] = rows_v[j, i, LANES:EMB] * SCALE
                return c2

            lax.fori_loop(0, CH, scale_row, 0, unroll=2)
            pltpu.sync_copy(rows_v, out_hbm.at[pl.ds(base, NCH)])
            return carry

        lax.fori_loop(0, n_groups, group, 0)

    return pl.kernel(
        body,
        out_type=jax.ShapeDtypeStruct((n_rows128, CH, EMB), jnp.float32),
        mesh=mesh,
        scratch_types=[
            pltpu.VMEM((NCH, CH), jnp.int32),
            pltpu.VMEM((NCH, CH, EMB), jnp.float32),
            pltpu.SemaphoreType.DMA,
        ],
        compiler_params=pltpu.CompilerParams(use_tc_tiling_on_sc=False),
    )


def kernel(tokens, table):
    n_tok = tokens.shape[0] * tokens.shape[1]
    assert n_tok % CH == 0
    n_rows128 = n_tok // CH
    nc, ns = 2, 16
    assert n_rows128 % (nc * ns * NCH) == 0
    idx = tokens.astype(jnp.int32).reshape(n_rows128, CH)
    out = _build(n_rows128, nc, ns)(idx, table)
    return out.reshape(tokens.shape[0], tokens.shape[1], EMB)


# speedup vs baseline: 1.2328x; 1.0380x over previous
import math

import jax
import jax.numpy as jnp
from jax import lax
from jax.experimental import pallas as pl
from jax.experimental.pallas import tpu as pltpu
from jax.experimental.pallas import tpu_sc as plsc

EMB = 32
SCALE = math.sqrt(EMB)
LANES = 16
CH = 128
NCH = 5
NBUF = 4


def _build(n_rows128, nc, ns):
    nw = nc * ns
    rpw = n_rows128 // nw
    ng = rpw // NCH
    assert ng % NBUF == 0 and ng >= NBUF

    mesh = plsc.VectorSubcoreMesh(core_axis_name="c", subcore_axis_name="s",
                                  num_cores=nc, num_subcores=ns)

    def body(idx_hbm, table_hbm, out_hbm, *scratch):
        iv = scratch[0:NBUF]
        rv = scratch[NBUF:2 * NBUF]
        gs = scratch[2 * NBUF:3 * NBUF]
        ss = scratch[3 * NBUF:4 * NBUF]
        w = lax.axis_index("s") * nc + lax.axis_index("c")
        row0 = w * rpw

        def load_and_fire(g, b):
            base = row0 + g * NCH
            pltpu.sync_copy(idx_hbm.at[pl.ds(base, NCH)], iv[b])
            for j in range(NCH):
                pltpu.async_copy(table_hbm.at[iv[b].at[j]], rv[b].at[j], gs[b])

        def wait_gather(g, b):
            base = row0 + g * NCH
            pltpu.make_async_copy(out_hbm.at[pl.ds(base, NCH)], rv[b], gs[b]).wait()

        def fire_scatter(g, b):
            base = row0 + g * NCH
            pltpu.async_copy(rv[b], out_hbm.at[pl.ds(base, NCH)], ss[b])

        def wait_scatter(g, b):
            base = row0 + g * NCH
            pltpu.make_async_copy(rv[b], out_hbm.at[pl.ds(base, NCH)], ss[b]).wait()

        def scale(b):
            r = rv[b]

            @pl.loop(0, CH, unroll=4)
            def _(i):
                for j in range(NCH):
                    r[j, i, 0:LANES] = r[j, i, 0:LANES] * SCALE
                    r[j, i, LANES:EMB] = r[j, i, LANES:EMB] * SCALE

        load_and_fire(0, 0)
        load_and_fire(1, 1)

        def round_(r_, carry):
            for k in range(NBUF):
                g = NBUF * r_ + k
                wait_gather(g, k)
                scale(k)
                fire_scatter(g, k)
                b2 = (k + 2) % NBUF

                @pl.when(g >= 2)
                def _():
                    wait_scatter(g - 2, b2)

                @pl.when(g + 2 < ng)
                def _():
                    load_and_fire(g + 2, b2)
            return carry

        lax.fori_loop(0, ng // NBUF, round_, 0)
        wait_scatter(ng - 2, (ng - 2) % NBUF)
        wait_scatter(ng - 1, (ng - 1) % NBUF)

    scratch = (
        [pltpu.VMEM((NCH, CH), jnp.int32) for _ in range(NBUF)]
        + [pltpu.VMEM((NCH, CH, EMB), jnp.float32) for _ in range(NBUF)]
        + [pltpu.SemaphoreType.DMA for _ in range(2 * NBUF)]
    )
    return pl.kernel(
        body,
        out_type=jax.ShapeDtypeStruct((n_rows128, CH, EMB), jnp.float32),
        mesh=mesh,
        scratch_types=scratch,
        compiler_params=pltpu.CompilerParams(use_tc_tiling_on_sc=False),
    )


def kernel(tokens, table):
    n_tok = tokens.shape[0] * tokens.shape[1]
    assert n_tok % CH == 0
    n_rows128 = n_tok // CH
    nc, ns = 2, 16
    assert n_rows128 % (nc * ns * NCH) == 0
    idx = tokens.astype(jnp.int32).reshape(n_rows128, CH)
    out = _build(n_rows128, nc, ns)(idx, table)
    return out.reshape(tokens.shape[0], tokens.shape[1], EMB)


# speedup vs baseline: 1.3473x; 1.0928x over previous
import math

import jax
import jax.numpy as jnp
from jax import lax
from jax.experimental import pallas as pl
from jax.experimental.pallas import tpu as pltpu
from jax.experimental.pallas import tpu_sc as plsc

EMB = 32
SCALE = math.sqrt(EMB)
LANES = 16
CH = 128
NBUF = 4


def _build(n_chunks, n_seq, n_batch, nc, ns):
    nw = nc * ns
    cpw = n_chunks // nw
    assert cpw % NBUF == 0 and cpw >= 2 * NBUF
    ib_bits = (n_batch // CH).bit_length() - 1

    mesh = plsc.VectorSubcoreMesh(core_axis_name="c", subcore_axis_name="s",
                                  num_cores=nc, num_subcores=ns)

    def body(idx_hbm, table_hbm, out_hbm, idx_all, *scratch):
        rows = scratch[0:NBUF]
        trans = scratch[NBUF:2 * NBUF]
        gs = scratch[2 * NBUF:3 * NBUF]
        ss = scratch[3 * NBUF:4 * NBUF]
        w = lax.axis_index("s") * nc + lax.axis_index("c")
        row0 = w * cpw
        pltpu.sync_copy(idx_hbm.at[pl.ds(row0, cpw)], idx_all)

        lane = lax.iota(jnp.int32, LANES)
        rowsel = [lane + (LANES * k) for k in range(CH // LANES)]

        def dst_slab(c):
            g = row0 + c
            j = lax.shift_right_logical(g, ib_bits)
            i0 = lax.shift_left(lax.bitwise_and(g, (1 << ib_bits) - 1), 7)
            return out_hbm.at[j, :, pl.ds(pl.multiple_of(i0, CH), CH)]

        def fire_gather(c, b):
            pltpu.async_copy(table_hbm.at[idx_all.at[c]], rows[b], gs[b])

        def wait_gather(b):
            pltpu.make_async_copy(table_hbm.at[pl.ds(0, CH)], rows[b], gs[b]).wait()

        def transpose_scale(b):
            r, t = rows[b], trans[b]

            @pl.loop(0, EMB, unroll=4)
            def _(e):
                col = jnp.full((LANES,), e, jnp.int32)
                for k in range(CH // LANES):
                    v = plsc.load_gather(r, [rowsel[k], col])
                    t[e, LANES * k:LANES * (k + 1)] = v * SCALE

        def fire_scatter(c, b):
            pltpu.async_copy(trans[b], dst_slab(c), ss[b])

        def wait_scatter(c, b):
            pltpu.make_async_copy(trans[b], dst_slab(c), ss[b]).wait()

        fire_gather(0, 0)
        fire_gather(1, 1)

        def round_(r_, carry):
            for k in range(NBUF):
                c = NBUF * r_ + k

                @pl.when(c + 2 < cpw)
                def _():
                    fire_gather(c + 2, (k + 2) % NBUF)

                wait_gather(k)

                @pl.when(c >= NBUF)
                def _():
                    wait_scatter(c - NBUF, k)

                transpose_scale(k)
                fire_scatter(c, k)
            return carry

        lax.fori_loop(0, cpw // NBUF, round_, 0)
        for k in range(NBUF):
            wait_scatter(cpw - NBUF + k, k)

    scratch = (
        [pltpu.VMEM((CH, EMB), jnp.float32) for _ in range(NBUF)]
        + [pltpu.VMEM((EMB, CH), jnp.float32) for _ in range(NBUF)]
        + [pltpu.SemaphoreType.DMA for _ in range(2 * NBUF)]
    )
    return pl.kernel(
        body,
        out_type=jax.ShapeDtypeStruct((n_seq, EMB, n_batch), jnp.float32),
        mesh=mesh,
        scratch_types=[pltpu.VMEM((cpw, CH), jnp.int32)] + scratch,
        compiler_params=pltpu.CompilerParams(use_tc_tiling_on_sc=False,
                                             needs_layout_passes=False),
    )


def kernel(tokens, table):
    n_batch, n_seq = tokens.shape
    n_chunks = n_batch * n_seq // CH
    assert n_batch % CH == 0 and (n_batch // CH) & (n_batch // CH - 1) == 0
    nc, ns = 2, 16
    assert n_chunks % (nc * ns * NBUF) == 0
    idx = jnp.maximum(tokens.T.reshape(n_chunks, CH), 0)
    out = _build(n_chunks, n_seq, n_batch, nc, ns)(idx, table)
    return out.transpose(2, 0, 1)


# speedup vs baseline: 1.8696x; 1.3877x over previous
import math

import jax
import jax.numpy as jnp
from jax import lax
from jax.experimental import pallas as pl
from jax.experimental.pallas import tpu as pltpu
from jax.experimental.pallas import tpu_sc as plsc

EMB = 32
SCALE = math.sqrt(EMB)
LANES = 16
CH = 128
NBUF = 8
DEPTH = 6


def _build(n_chunks, n_seq, n_batch, nc, ns):
    nw = nc * ns
    cpw = n_chunks // nw
    assert cpw % NBUF == 0 and cpw >= 2 * NBUF
    ib_bits = (n_batch // CH).bit_length() - 1

    mesh = plsc.VectorSubcoreMesh(core_axis_name="c", subcore_axis_name="s",
                                  num_cores=nc, num_subcores=ns)

    def body(idx_hbm, table_hbm, out_hbm, idx_all, *scratch):
        rows = scratch[0:NBUF]
        trans = scratch[NBUF:2 * NBUF]
        gs = scratch[2 * NBUF:3 * NBUF]
        ss = scratch[3 * NBUF:4 * NBUF]
        w = lax.axis_index("s") * nc + lax.axis_index("c")
        row0 = w * cpw
        pltpu.sync_copy(idx_hbm.at[pl.ds(row0, cpw)], idx_all)

        lane = lax.iota(jnp.int32, LANES)
        rowsel = [lane + (LANES * k) for k in range(CH // LANES)]

        def dst_slab(c):
            g = row0 + c
            j = lax.shift_right_logical(g, ib_bits)
            i0 = lax.shift_left(lax.bitwise_and(g, (1 << ib_bits) - 1), 7)
            return out_hbm.at[j, :, pl.ds(pl.multiple_of(i0, CH), CH)]

        def fire_gather(c, b):
            pltpu.async_copy(table_hbm.at[idx_all.at[c]], rows[b], gs[b])

        def wait_gather(b):
            pltpu.make_async_copy(table_hbm.at[pl.ds(0, CH)], rows[b],
                                  gs[b]).wait()

        def transpose_scale(b):
            r, t = rows[b], trans[b]

            @pl.loop(0, EMB, unroll=4)
            def _(e):
                colrot = lax.bitwise_and(lane + e, EMB - 1)
                for k in range(CH // LANES):
                    v = plsc.load_gather(r, [rowsel[k], colrot])
                    plsc.store_scatter(t, [colrot, rowsel[k]], v * SCALE)

        def fire_scatter(c, b):
            pltpu.async_copy(trans[b], dst_slab(c), ss[b])

        def wait_scatter(c, b):
            pltpu.make_async_copy(trans[b], dst_slab(c), ss[b]).wait()

        for b in range(DEPTH):
            fire_gather(b, b)

        def round_(r_, carry):
            for k in range(NBUF):
                c = NBUF * r_ + k

                @pl.when(c + DEPTH < cpw)
                def _():
                    fire_gather(c + DEPTH, (k + DEPTH) % NBUF)

                wait_gather(k)

                @pl.when(c >= NBUF)
                def _():
                    wait_scatter(c - NBUF, k)

                transpose_scale(k)
                fire_scatter(c, k)
            return carry

        lax.fori_loop(0, cpw // NBUF, round_, 0)
        for k in range(NBUF):
            wait_scatter(cpw - NBUF + k, k)

    scratch = (
        [pltpu.VMEM((CH, EMB), jnp.float32) for _ in range(NBUF)]
        + [pltpu.VMEM((EMB, CH), jnp.float32) for _ in range(NBUF)]
        + [pltpu.SemaphoreType.DMA for _ in range(2 * NBUF)]
    )
    return pl.kernel(
        body,
        out_type=jax.ShapeDtypeStruct((n_seq, EMB, n_batch), jnp.float32),
        mesh=mesh,
        scratch_types=[pltpu.VMEM((cpw, CH), jnp.int32)] + scratch,
        compiler_params=pltpu.CompilerParams(use_tc_tiling_on_sc=False,
                                             needs_layout_passes=False),
    )


def kernel(tokens, table):
    n_batch, n_seq = tokens.shape
    n_chunks = n_batch * n_seq // CH
    assert n_batch % CH == 0 and (n_batch // CH) & (n_batch // CH - 1) == 0
    nc, ns = 2, 16
    assert n_chunks % (nc * ns * NBUF) == 0
    idx = jnp.maximum(tokens.T.reshape(n_chunks, CH), 0)
    out = _build(n_chunks, n_seq, n_batch, nc, ns)(idx, table)
    return out.transpose(2, 0, 1)
